# Initial kernel scaffold; baseline (speedup 1.0000x reference)
import jax
import jax.numpy as jnp
from jax.experimental import pallas as pl


def kernel(node_in, bb, rigids_7, res_mask, edge_index, params):
    raise NotImplementedError("write your pallas kernel here")


# speedup vs baseline: 1.2544x; 1.2544x over previous
import functools
import numpy as np
import jax
import jax.numpy as jnp
from jax.experimental import pallas as pl

HI = jax.lax.Precision.HIGHEST

NUM_RBF = 16
N_PT = 8


def _tile(n, prefs=(2000, 1000, 500, 250, 200, 100, 40, 8)):
    for t in prefs:
        if n % t == 0 and t % 8 == 0:
            return t
    return n


def _edge_geom_consts():
    M1 = np.zeros((16, 80), np.float32)
    M2 = np.zeros((16, 80), np.float32)
    M3 = np.zeros((80, 32), np.float32)
    for a in range(5):
        for b in range(5):
            for c in range(3):
                q = (a * 5 + b) * 3 + c
                M1[3 * a + c, q] = 1.0
                M2[3 * b + c, q] = 1.0
                M3[q, a * 5 + b] = 1.0
    S = np.zeros((32, 416), np.float32)
    for p in range(25):
        for k in range(16):
            S[p, 16 * p + k] = 1.0
    mu = np.linspace(2.0, 22.0, NUM_RBF).astype(np.float32)
    MU = np.zeros((416,), np.float32)
    for p in range(25):
        MU[16 * p:16 * p + 16] = mu
    SIG = np.ones((416,), np.float32) * ((22.0 - 2.0) / NUM_RBF)
    PH = np.zeros((16,), np.float32)
    PH[8:] = np.pi / 2
    return M1, M2, M3, S, MU, SIG, PH


_M1, _M2, _M3, _S, _MU, _SIG, _PH = _edge_geom_consts()

_M4 = np.zeros((32, 16), np.float32)
for _p in range(N_PT):
    for _c in range(3):
        _M4[3 * _p + _c, _p] = 1.0

_MP = np.zeros((24, 72), np.float32)
_MR = np.zeros((16, 72), np.float32)
_MS = np.zeros((72, 32), np.float32)
_MT = np.zeros((16, 32), np.float32)
for _p in range(N_PT):
    for _i in range(3):
        for _j in range(3):
            _q = 9 * _p + 3 * _i + _j
            _MP[3 * _p + _j, _q] = 1.0
            _MR[3 * _i + _j, _q] = 1.0
            _MS[_q, 3 * _p + _i] = 1.0
        _MT[9 + _i, 3 * _p + _i] = 1.0


def _ln(x, g, b, eps=1e-5):
    m = jnp.mean(x, axis=-1, keepdims=True)
    xc = x - m
    v = jnp.mean(xc * xc, axis=-1, keepdims=True)
    return xc / jnp.sqrt(v + eps) * g + b


def _edge_embed_body(ebs, ebd, ang, m1, m2, m3, ssel, muv, ph,
                     w0a, w0p, b0, w1, b1, w2, b2, g, beta, z_out):
    t1 = jnp.dot(ebs[...], m1[...], precision=HI)
    t2 = jnp.dot(ebd[...], m2[...], precision=HI)
    diff = t1 - t2 + 1e-8
    d2 = jnp.dot(diff * diff, m3[...], precision=HI)
    d = jnp.sqrt(d2)
    dbig = jnp.dot(d, ssel[...], precision=HI)
    x = jnp.exp(-(((dbig - muv[...]) * 0.8) ** 2))
    pe = jnp.cos(ang[...] - ph[...])
    h = jax.nn.relu(jnp.dot(x, w0a[...]) + jnp.dot(pe, w0p[...]) + b0[...])
    h = jax.nn.relu(jnp.dot(h, w1[...]) + b1[...])
    z_out[...] = _ln(jnp.dot(h, w2[...]) + b2[...], g[...], beta[...])


def _message_body(gsp, gdp, asrc, adst, z, m4, wmz, wmp, bm, wm2, bm2, msg_out):
    diff = gdp[...] - gsp[...] + 1e-8
    pd2 = jnp.dot(diff * diff, m4[...], precision=HI)
    pd = jnp.sqrt(pd2)
    m = jax.nn.relu(asrc[...] + adst[...] + jnp.dot(z[...], wmz[...])
                    + jnp.dot(pd, wmp[...]) + bm[...])
    msg_out[...] = jnp.dot(m, wm2[...]) + bm2[...]


def _edge_update_body(esrc, edst, z, wez, be, g, beta, z_out):
    eupd = jax.nn.relu(edst[...] + esrc[...] + jnp.dot(z[...], wez[...]) + be[...])
    z_out[...] = _ln(z[...] + eupd, g[...], beta[...])


def _node_prep_body(ni, bbf, rig, w0, b0, w1, b1, w2, b2, g, beta,
                    s_out, bb5_out, rt_out):
    h = jax.nn.relu(jnp.dot(ni[...], w0[...]) + b0[...])
    h = jax.nn.relu(jnp.dot(h, w1[...]) + b1[...])
    s_out[...] = _ln(jnp.dot(h, w2[...]) + b2[...], g[...], beta[...])
    bf = bbf[...]
    b = bf[:, 3:6] - bf[:, 0:3]
    cv = bf[:, 6:9] - bf[:, 3:6]
    a = jnp.concatenate([
        b[:, 1:2] * cv[:, 2:3] - b[:, 2:3] * cv[:, 1:2],
        b[:, 2:3] * cv[:, 0:1] - b[:, 0:1] * cv[:, 2:3],
        b[:, 0:1] * cv[:, 1:2] - b[:, 1:2] * cv[:, 0:1],
    ], axis=1)
    cb = -0.58273431 * a + 0.56802827 * b - 0.54067466 * cv + bf[:, 3:6]
    zcol = jnp.zeros_like(cb[:, 0:1])
    bb5_out[...] = jnp.concatenate([bf, cb, zcol], axis=1)
    q = rig[:, 0:4]
    qn = q / (jnp.sqrt(jnp.sum(q * q, axis=1, keepdims=True)) + 1e-8)
    w = qn[:, 0:1]; x = qn[:, 1:2]; y = qn[:, 2:3]; zc = qn[:, 3:4]
    rt_out[...] = jnp.concatenate([
        1 - 2 * (y * y + zc * zc), 2 * (x * y - zc * w), 2 * (x * zc + y * w),
        2 * (x * y + zc * w), 1 - 2 * (x * x + zc * zc), 2 * (y * zc - x * w),
        2 * (x * zc - y * w), 2 * (y * zc + x * w), 1 - 2 * (x * x + y * y),
        rig[:, 4:7], zcol, zcol, zcol, zcol,
    ], axis=1)


def _node_pre_body(s, rt, mp, mr, ms, mt, wp, bp, wms, wmd,
                   gpt_out, ams_out, amd_out):
    pts = jnp.dot(s[...], wp[...]) + bp[...]
    pe = jnp.dot(pts, mp[...], precision=HI)
    re = jnp.dot(rt[...], mr[...], precision=HI)
    gpf = jnp.dot(pe * re, ms[...], precision=HI) + jnp.dot(rt[...], mt[...], precision=HI)
    gpt_out[...] = gpf
    ams_out[...] = jnp.dot(s[...], wms[...])
    amd_out[...] = jnp.dot(s[...], wmd[...])


def _node_post_body(s, agg, deg8, mask8, wns, wna, bn, g, beta, wes, wed,
                    s_out, aes_out, aed_out):
    deg = jnp.clip(deg8[:, 0:1], 1.0, None)
    a = agg[...] / deg
    upd = jax.nn.relu(jnp.dot(s[...], wns[...]) + jnp.dot(a, wna[...]) + bn[...])
    s2 = _ln(s[...] + upd, g[...], beta[...]) * mask8[:, 0:1]
    s_out[...] = s2
    aes_out[...] = jnp.dot(s2, wes[...])
    aed_out[...] = jnp.dot(s2, wed[...])


def _final_body(s, wmu, bmu, wlv, blv, mu_out, lv_out):
    mu_out[...] = jnp.dot(s[...], wmu[...]) + bmu[...]
    lv_out[...] = jnp.dot(s[...], wlv[...]) + blv[...]


def _row_call(body, n_rows, tile, row_widths, const_shapes, out_widths, args):
    grid = (n_rows // tile,)
    in_specs = [pl.BlockSpec((tile, w), lambda i: (i, 0)) for w in row_widths]
    in_specs += [pl.BlockSpec(s, lambda i, _r=len(s): (0,) * _r) for s in const_shapes]
    out_specs = [pl.BlockSpec((tile, w), lambda i: (i, 0)) for w in out_widths]
    out_shape = [jax.ShapeDtypeStruct((n_rows, w), jnp.float32) for w in out_widths]
    if len(out_widths) == 1:
        out_specs = out_specs[0]
        out_shape = out_shape[0]
    return pl.pallas_call(
        body, grid=grid, in_specs=in_specs, out_specs=out_specs,
        out_shape=out_shape)(*args)


def kernel(node_in, bb, rigids_7, res_mask, edge_index, params):
    p = params
    src = edge_index[1]
    dst = edge_index[0]
    n = node_in.shape[0]
    e = src.shape[0]
    tn = _tile(n)
    te = _tile(e)
    f32 = jnp.float32

    bbf = bb.reshape(n, 12)
    rig = rigids_7
    w0p_shapes = [(node_in.shape[1], 512), (1, 512), (512, 512), (1, 512),
                  (512, 256), (1, 256), (1, 256), (1, 256)]
    s0, bb5f, rt = _row_call(
        _node_prep_body, n, tn, [node_in.shape[1], 12, 7], w0p_shapes,
        [256, 16, 16],
        (node_in, bbf, rig,
         p['en_W0'], p['en_b0'].reshape(1, -1), p['en_W1'], p['en_b1'].reshape(1, -1),
         p['en_W2'], p['en_b2'].reshape(1, -1), p['en_g'].reshape(1, -1),
         p['en_beta'].reshape(1, -1)))

    ebs = jnp.take(bb5f, src, axis=0)
    ebd = jnp.take(bb5f, dst, axis=0)
    dif = (dst - src).astype(f32)
    freq = jnp.exp(jnp.arange(0, 16, 2, dtype=f32) * (-np.log(10000.0) / 16))
    freq2 = jnp.concatenate([freq, freq])
    ang = dif[:, None] * freq2[None, :]
    w0a = jnp.concatenate([p['ee_W0'][:400], jnp.zeros((16, 256), f32)], axis=0)
    w0pe = p['ee_W0'][400:416]
    z = _row_call(
        _edge_embed_body, e, te, [16, 16, 16],
        [(16, 80), (16, 80), (80, 32), (32, 416), (1, 416), (1, 16),
         (416, 256), (16, 256), (1, 256), (256, 256), (1, 256), (256, 128),
         (1, 128), (1, 128), (1, 128)],
        [128],
        (ebs, ebd, ang, jnp.asarray(_M1), jnp.asarray(_M2), jnp.asarray(_M3),
         jnp.asarray(_S), jnp.asarray(_MU).reshape(1, -1),
         jnp.asarray(_PH).reshape(1, -1),
         w0a, w0pe, p['ee_b0'].reshape(1, -1), p['ee_W1'],
         p['ee_b1'].reshape(1, -1), p['ee_W2'], p['ee_b2'].reshape(1, -1),
         p['ee_g'].reshape(1, -1), p['ee_beta'].reshape(1, -1)))

    ones_e = jnp.ones((e,), f32)
    deg = jax.ops.segment_sum(ones_e, dst, num_segments=n)
    deg8 = jnp.broadcast_to(deg[:, None], (n, 8))
    mask8 = jnp.broadcast_to(res_mask[:, None], (n, 8))

    s = s0
    num_layers = 2
    for l in range(num_layers):
        wp24 = p[f'Wp{l}']
        gpt, ams, amd = _row_call(
            _node_pre_body, n, tn, [256, 16],
            [(24, 72), (16, 72), (72, 32), (16, 32),
             (256, 24), (1, 24), (256, 128), (256, 128)], [32, 128, 128],
            (s, rt, jnp.asarray(_MP), jnp.asarray(_MR), jnp.asarray(_MS),
             jnp.asarray(_MT), wp24, p[f'bp{l}'].reshape(1, -1),
             p[f'Wms{l}'], p[f'Wmd{l}']))
        gsp = jnp.take(gpt, src, axis=0)
        gdp = jnp.take(gpt, dst, axis=0)
        asrc = jnp.take(ams, src, axis=0)
        adst = jnp.take(amd, dst, axis=0)
        wmp16 = jnp.concatenate([p[f'Wmp{l}'], jnp.zeros((8, 128), f32)], axis=0)
        msg = _row_call(
            _message_body, e, te, [32, 32, 128, 128, 128],
            [(32, 16), (128, 128), (16, 128), (1, 128), (128, 256), (1, 256)], [256],
            (gsp, gdp, asrc, adst, z, jnp.asarray(_M4), p[f'Wmz{l}'], wmp16,
             p[f'bm{l}'].reshape(1, -1), p[f'Wm2{l}'], p[f'bm2{l}'].reshape(1, -1)))
        agg = jax.ops.segment_sum(msg, dst, num_segments=n)
        wn = p[f'Wn{l}']
        s, aes, aed = _row_call(
            _node_post_body, n, tn, [256, 256, 8, 8],
            [(256, 256), (256, 256), (1, 256), (1, 256), (1, 256),
             (256, 128), (256, 128)],
            [256, 128, 128],
            (s, agg, deg8, mask8, wn[:256], wn[256:], p[f'bn{l}'].reshape(1, -1),
             p[f'lns_g{l}'].reshape(1, -1), p[f'lns_b{l}'].reshape(1, -1),
             p[f'Wes{l}'], p[f'Wed{l}']))
        if l < num_layers - 1:
            esrc = jnp.take(aes, src, axis=0)
            edst = jnp.take(aed, dst, axis=0)
            z = _row_call(
                _edge_update_body, e, te, [128, 128, 128],
                [(128, 128), (1, 128), (1, 128), (1, 128)], [128],
                (esrc, edst, z, p[f'Wez{l}'], p[f'be{l}'].reshape(1, -1),
                 p[f'lnz_g{l}'].reshape(1, -1), p[f'lnz_b{l}'].reshape(1, -1)))

    mu, logvar = _row_call(
        _final_body, n, tn, [256],
        [(256, 256), (1, 256), (256, 256), (1, 256)], [256, 256],
        (s, p['Wmu'], p['bmu'].reshape(1, -1), p['Wlv'], p['blv'].reshape(1, -1)))
    return mu, logvar


# speedup vs baseline: 3.4636x; 2.7612x over previous
import functools
import numpy as np
import jax
import jax.numpy as jnp
from jax import lax
from jax.experimental import pallas as pl
from jax.experimental.pallas import tpu as pltpu
from jax.experimental.pallas import tpu_sc as plsc

_NC = 2
_NS = 16
_NW = _NC * _NS
_CH = 200

HI = jax.lax.Precision.HIGHEST

NUM_RBF = 16
N_PT = 8


def _tile(n, prefs=(2000, 1000, 500, 250, 200, 100, 40, 8)):
    for t in prefs:
        if n % t == 0 and t % 8 == 0:
            return t
    return n


def _edge_geom_consts():
    M1 = np.zeros((16, 80), np.float32)
    M2 = np.zeros((16, 80), np.float32)
    M3 = np.zeros((80, 32), np.float32)
    for a in range(5):
        for b in range(5):
            for c in range(3):
                q = (a * 5 + b) * 3 + c
                M1[3 * a + c, q] = 1.0
                M2[3 * b + c, q] = 1.0
                M3[q, a * 5 + b] = 1.0
    S = np.zeros((32, 416), np.float32)
    for p in range(25):
        for k in range(16):
            S[p, 16 * p + k] = 1.0
    mu = np.linspace(2.0, 22.0, NUM_RBF).astype(np.float32)
    MU = np.zeros((416,), np.float32)
    for p in range(25):
        MU[16 * p:16 * p + 16] = mu
    SIG = np.ones((416,), np.float32) * ((22.0 - 2.0) / NUM_RBF)
    PH = np.zeros((16,), np.float32)
    PH[8:] = np.pi / 2
    return M1, M2, M3, S, MU, SIG, PH


_M1, _M2, _M3, _S, _MU, _SIG, _PH = _edge_geom_consts()

_M4 = np.zeros((32, 16), np.float32)
for _p in range(N_PT):
    for _c in range(3):
        _M4[3 * _p + _c, _p] = 1.0

_MP = np.zeros((24, 72), np.float32)
_MR = np.zeros((16, 72), np.float32)
_MS = np.zeros((72, 32), np.float32)
_MT = np.zeros((16, 32), np.float32)
for _p in range(N_PT):
    for _i in range(3):
        for _j in range(3):
            _q = 9 * _p + 3 * _i + _j
            _MP[3 * _p + _j, _q] = 1.0
            _MR[3 * _i + _j, _q] = 1.0
            _MS[_q, 3 * _p + _i] = 1.0
        _MT[9 + _i, 3 * _p + _i] = 1.0


def _ln(x, g, b, eps=1e-5):
    m = jnp.mean(x, axis=-1, keepdims=True)
    xc = x - m
    v = jnp.mean(xc * xc, axis=-1, keepdims=True)
    return xc / jnp.sqrt(v + eps) * g + b


def _edge_embed_body(ebs, ebd, ang, m1, m2, m3, ssel, muv, ph,
                     w0a, w0p, b0, w1, b1, w2, b2, g, beta, z_out):
    t1 = jnp.dot(ebs[...], m1[...], precision=HI)
    t2 = jnp.dot(ebd[...], m2[...], precision=HI)
    diff = t1 - t2 + 1e-8
    d2 = jnp.dot(diff * diff, m3[...], precision=HI)
    d = jnp.sqrt(d2)
    dbig = jnp.dot(d, ssel[...], precision=HI)
    x = jnp.exp(-(((dbig - muv[...]) * 0.8) ** 2))
    pe = jnp.cos(ang[...] - ph[...])
    h = jax.nn.relu(jnp.dot(x, w0a[...]) + jnp.dot(pe, w0p[...]) + b0[...])
    h = jax.nn.relu(jnp.dot(h, w1[...]) + b1[...])
    z_out[...] = _ln(jnp.dot(h, w2[...]) + b2[...], g[...], beta[...])


def _message_body(gsp, gdp, asrc, adst, z, m4, wmz, wmp, bm, wm2, bm2,
                  msg0_out, msg1_out):
    diff = gdp[...] - gsp[...] + 1e-8
    pd2 = jnp.dot(diff * diff, m4[...], precision=HI)
    pd = jnp.sqrt(pd2)
    m = jax.nn.relu(asrc[...] + adst[...] + jnp.dot(z[...], wmz[...])
                    + jnp.dot(pd, wmp[...]) + bm[...])
    msg = jnp.dot(m, wm2[...]) + bm2[...]
    msg0_out[...] = msg[:, 0:128]
    msg1_out[...] = msg[:, 128:256]


def _edge_update_body(esrc, edst, z, wez, be, g, beta, z_out):
    eupd = jax.nn.relu(edst[...] + esrc[...] + jnp.dot(z[...], wez[...]) + be[...])
    z_out[...] = _ln(z[...] + eupd, g[...], beta[...])


def _node_prep_body(ni, bbf, rig, w0, b0, w1, b1, w2, b2, g, beta,
                    s_out, bb5_out, rt_out):
    h = jax.nn.relu(jnp.dot(ni[...], w0[...]) + b0[...])
    h = jax.nn.relu(jnp.dot(h, w1[...]) + b1[...])
    s_out[...] = _ln(jnp.dot(h, w2[...]) + b2[...], g[...], beta[...])
    bf = bbf[...]
    b = bf[:, 3:6] - bf[:, 0:3]
    cv = bf[:, 6:9] - bf[:, 3:6]
    a = jnp.concatenate([
        b[:, 1:2] * cv[:, 2:3] - b[:, 2:3] * cv[:, 1:2],
        b[:, 2:3] * cv[:, 0:1] - b[:, 0:1] * cv[:, 2:3],
        b[:, 0:1] * cv[:, 1:2] - b[:, 1:2] * cv[:, 0:1],
    ], axis=1)
    cb = -0.58273431 * a + 0.56802827 * b - 0.54067466 * cv + bf[:, 3:6]
    zcol = jnp.zeros_like(cb[:, 0:1])
    bb5_out[...] = jnp.concatenate([bf, cb, zcol], axis=1)
    q = rig[:, 0:4]
    qn = q / (jnp.sqrt(jnp.sum(q * q, axis=1, keepdims=True)) + 1e-8)
    w = qn[:, 0:1]; x = qn[:, 1:2]; y = qn[:, 2:3]; zc = qn[:, 3:4]
    rt_out[...] = jnp.concatenate([
        1 - 2 * (y * y + zc * zc), 2 * (x * y - zc * w), 2 * (x * zc + y * w),
        2 * (x * y + zc * w), 1 - 2 * (x * x + zc * zc), 2 * (y * zc - x * w),
        2 * (x * zc - y * w), 2 * (y * zc + x * w), 1 - 2 * (x * x + y * y),
        rig[:, 4:7], zcol, zcol, zcol, zcol,
    ], axis=1)


def _node_pre_body(s, rt, mp, mr, ms, mt, wp, bp, wms, wmd,
                   gpt_out, ams_out, amd_out):
    pts = jnp.dot(s[...], wp[...]) + bp[...]
    pe = jnp.dot(pts, mp[...], precision=HI)
    re = jnp.dot(rt[...], mr[...], precision=HI)
    gpf = jnp.dot(pe * re, ms[...], precision=HI) + jnp.dot(rt[...], mt[...], precision=HI)
    gpt_out[...] = gpf
    ams_out[...] = jnp.dot(s[...], wms[...])
    amd_out[...] = jnp.dot(s[...], wmd[...])


def _node_post_body(s, agg0, agg1, deg16, mask8, wns, wna0, wna1, bn, g, beta,
                    wes, wed, s_out, aes_out, aed_out):
    deg = jnp.clip(deg16[:, 0:1], 1.0, None)
    upd = jax.nn.relu(jnp.dot(s[...], wns[...]) + jnp.dot(agg0[...] / deg, wna0[...])
                      + jnp.dot(agg1[...] / deg, wna1[...]) + bn[...])
    s2 = _ln(s[...] + upd, g[...], beta[...]) * mask8[:, 0:1]
    s_out[...] = s2
    aes_out[...] = jnp.dot(s2, wes[...])
    aed_out[...] = jnp.dot(s2, wed[...])


def _final_body(s, wmu, bmu, wlv, blv, mu_out, lv_out):
    mu_out[...] = jnp.dot(s[...], wmu[...]) + bmu[...]
    lv_out[...] = jnp.dot(s[...], wlv[...]) + blv[...]


def _sc_mesh():
    return plsc.VectorSubcoreMesh(core_axis_name="c", subcore_axis_name="s")


_SC_CP = pltpu.CompilerParams(use_tc_tiling_on_sc=False)


def _sc_gather_pair(table_a, idx_a, table_b, idx_b):
    e = idx_a.shape[0]
    d = table_a.shape[1]
    per_w = e // _NW
    nch = per_w // _CH

    @functools.partial(
        pl.kernel, mesh=_sc_mesh(), compiler_params=_SC_CP,
        out_type=[jax.ShapeDtypeStruct((e, d), jnp.float32),
                  jax.ShapeDtypeStruct((e, d), jnp.float32)],
        scratch_types=[pltpu.VMEM((_CH,), jnp.int32),
                       pltpu.VMEM((_CH, d), jnp.float32),
                       pltpu.VMEM((_CH,), jnp.int32),
                       pltpu.VMEM((_CH, d), jnp.float32),
                       pltpu.SemaphoreType.DMA,
                       pltpu.SemaphoreType.DMA],
    )
    def k(ta, ia, tb, ib, oa, ob, iva, rva, ivb, rvb, sema, semb):
        wid = lax.axis_index("s") * _NC + lax.axis_index("c")
        base = wid * per_w

        @pl.loop(0, nch)
        def _(i):
            b = base + i * _CH
            pltpu.sync_copy(ia.at[pl.ds(b, _CH)], iva)
            cpa = pltpu.async_copy(ta.at[iva], rva, sema)
            pltpu.sync_copy(ib.at[pl.ds(b, _CH)], ivb)
            cpb = pltpu.async_copy(tb.at[ivb], rvb, semb)
            cpa.wait()
            pltpu.sync_copy(rva, oa.at[pl.ds(b, _CH)])
            cpb.wait()
            pltpu.sync_copy(rvb, ob.at[pl.ds(b, _CH)])

    return k(table_a, idx_a, table_b, idx_b)


def _sc_segsum_halves(msg0, msg1, dst2d, zrows, n):
    e = msg0.shape[0]
    nch_tot = e // _CH
    nzch = n // _CH

    @functools.partial(
        pl.kernel, mesh=_sc_mesh(), compiler_params=_SC_CP,
        out_type=[jax.ShapeDtypeStruct((n, 128), jnp.float32),
                  jax.ShapeDtypeStruct((n, 128), jnp.float32)],
        scratch_types=[pltpu.VMEM((1, _CH), jnp.int32),
                       pltpu.VMEM((_CH, 128), jnp.float32),
                       pltpu.VMEM_SHARED((n, 128), jnp.float32),
                       pltpu.SemaphoreType.DMA],
    )
    def k(m0, m1, dd, zr, o0, o1, idxv, rows, acc, sem):
        cid = lax.axis_index("c")
        sid = lax.axis_index("s")

        @pl.loop(0, nzch)
        def _(i):
            @pl.when((i % _NS) == sid)
            def _():
                pltpu.sync_copy(zr, acc.at[pl.ds(i * _CH, _CH)])

        plsc.subcore_barrier()

        def scan(mref):
            @pl.loop(0, nch_tot)
            def _(i):
                @pl.when((i % _NS) == sid)
                def _():
                    pltpu.sync_copy(dd.at[pl.ds(i, 1)], idxv)
                    pltpu.sync_copy(mref.at[pl.ds(i * _CH, _CH)], rows)
                    pltpu.sync_copy(rows, acc.at[idxv.at[0]], add=True)

        @pl.when(cid == 0)
        def _():
            scan(m0)

        @pl.when(cid == 1)
        def _():
            scan(m1)

        plsc.subcore_barrier()

        def dump(oref):
            @pl.loop(0, nzch)
            def _(i):
                @pl.when((i % _NS) == sid)
                def _():
                    pltpu.sync_copy(acc.at[pl.ds(i * _CH, _CH)],
                                    oref.at[pl.ds(i * _CH, _CH)])

        @pl.when(cid == 0)
        def _():
            dump(o0)

        @pl.when(cid == 1)
        def _():
            dump(o1)

    return k(msg0, msg1, dst2d, zrows)


def _sc_deg(dst2d, ones16, z16, n):
    nch_tot = (dst2d.shape[0] * dst2d.shape[1]) // _CH
    nzch = n // _CH

    @functools.partial(
        pl.kernel, mesh=_sc_mesh(), compiler_params=_SC_CP,
        out_type=jax.ShapeDtypeStruct((n, 16), jnp.float32),
        scratch_types=[pltpu.VMEM((1, _CH), jnp.int32),
                       pltpu.VMEM((_CH, 16), jnp.float32),
                       pltpu.VMEM_SHARED((n, 16), jnp.float32),
                       pltpu.SemaphoreType.DMA],
    )
    def k(dd, ones_h, zr, out, idxv, onev, acc, sem):
        cid = lax.axis_index("c")
        sid = lax.axis_index("s")

        @pl.when(cid == 0)
        def _():
            pltpu.sync_copy(ones_h, onev)

            @pl.loop(0, nzch)
            def _(i):
                @pl.when((i % _NS) == sid)
                def _():
                    pltpu.sync_copy(zr, acc.at[pl.ds(i * _CH, _CH)])

            plsc.subcore_barrier()

            @pl.loop(0, nch_tot)
            def _(i):
                @pl.when((i % _NS) == sid)
                def _():
                    pltpu.sync_copy(dd.at[pl.ds(i, 1)], idxv)
                    pltpu.sync_copy(onev, acc.at[idxv.at[0]], add=True)

            plsc.subcore_barrier()

            @pl.loop(0, nzch)
            def _(i):
                @pl.when((i % _NS) == sid)
                def _():
                    pltpu.sync_copy(acc.at[pl.ds(i * _CH, _CH)],
                                    out.at[pl.ds(i * _CH, _CH)])

    return k(dst2d, ones16, z16)


def _row_call(body, n_rows, tile, row_widths, const_shapes, out_widths, args):
    grid = (n_rows // tile,)
    in_specs = [pl.BlockSpec((tile, w), lambda i: (i, 0)) for w in row_widths]
    in_specs += [pl.BlockSpec(s, lambda i, _r=len(s): (0,) * _r) for s in const_shapes]
    out_specs = [pl.BlockSpec((tile, w), lambda i: (i, 0)) for w in out_widths]
    out_shape = [jax.ShapeDtypeStruct((n_rows, w), jnp.float32) for w in out_widths]
    if len(out_widths) == 1:
        out_specs = out_specs[0]
        out_shape = out_shape[0]
    return pl.pallas_call(
        body, grid=grid, in_specs=in_specs, out_specs=out_specs,
        out_shape=out_shape)(*args)


def kernel(node_in, bb, rigids_7, res_mask, edge_index, params):
    p = params
    src = edge_index[1]
    dst = edge_index[0]
    n = node_in.shape[0]
    e = src.shape[0]
    tn = _tile(n)
    te = _tile(e)
    f32 = jnp.float32

    bbf = bb.reshape(n, 12)
    rig = rigids_7
    w0p_shapes = [(node_in.shape[1], 512), (1, 512), (512, 512), (1, 512),
                  (512, 256), (1, 256), (1, 256), (1, 256)]
    s0, bb5f, rt = _row_call(
        _node_prep_body, n, tn, [node_in.shape[1], 12, 7], w0p_shapes,
        [256, 16, 16],
        (node_in, bbf, rig,
         p['en_W0'], p['en_b0'].reshape(1, -1), p['en_W1'], p['en_b1'].reshape(1, -1),
         p['en_W2'], p['en_b2'].reshape(1, -1), p['en_g'].reshape(1, -1),
         p['en_beta'].reshape(1, -1)))

    use_sc = (e % (_NW * _CH) == 0) and (n % _CH == 0) and (n * 128 * 4 <= 6 * 2 ** 20)
    if use_sc:
        dst2d = dst.reshape(e // _CH, _CH)
        zrows = jnp.zeros((_CH, 128), f32)
        zrows16 = jnp.zeros((_CH, 16), f32)
        ones16 = jnp.concatenate([jnp.ones((_CH, 1), f32),
                                  jnp.zeros((_CH, 15), f32)], axis=1)
        deg16 = _sc_deg(dst2d, ones16, zrows16, n)
        ebs, ebd = _sc_gather_pair(bb5f, src, bb5f, dst)
    else:
        deg16 = jnp.broadcast_to(
            jax.ops.segment_sum(jnp.ones((e,), f32), dst, num_segments=n)[:, None],
            (n, 16))
        ebs = jnp.take(bb5f, src, axis=0)
        ebd = jnp.take(bb5f, dst, axis=0)
    dif = (dst - src).astype(f32)
    freq = jnp.exp(jnp.arange(0, 16, 2, dtype=f32) * (-np.log(10000.0) / 16))
    freq2 = jnp.concatenate([freq, freq])
    ang = dif[:, None] * freq2[None, :]
    w0a = jnp.concatenate([p['ee_W0'][:400], jnp.zeros((16, 256), f32)], axis=0)
    w0pe = p['ee_W0'][400:416]
    z = _row_call(
        _edge_embed_body, e, te, [16, 16, 16],
        [(16, 80), (16, 80), (80, 32), (32, 416), (1, 416), (1, 16),
         (416, 256), (16, 256), (1, 256), (256, 256), (1, 256), (256, 128),
         (1, 128), (1, 128), (1, 128)],
        [128],
        (ebs, ebd, ang, jnp.asarray(_M1), jnp.asarray(_M2), jnp.asarray(_M3),
         jnp.asarray(_S), jnp.asarray(_MU).reshape(1, -1),
         jnp.asarray(_PH).reshape(1, -1),
         w0a, w0pe, p['ee_b0'].reshape(1, -1), p['ee_W1'],
         p['ee_b1'].reshape(1, -1), p['ee_W2'], p['ee_b2'].reshape(1, -1),
         p['ee_g'].reshape(1, -1), p['ee_beta'].reshape(1, -1)))

    mask8 = jnp.broadcast_to(res_mask[:, None], (n, 8))

    s = s0
    num_layers = 2
    for l in range(num_layers):
        wp24 = p[f'Wp{l}']
        gpt, ams, amd = _row_call(
            _node_pre_body, n, tn, [256, 16],
            [(24, 72), (16, 72), (72, 32), (16, 32),
             (256, 24), (1, 24), (256, 128), (256, 128)], [32, 128, 128],
            (s, rt, jnp.asarray(_MP), jnp.asarray(_MR), jnp.asarray(_MS),
             jnp.asarray(_MT), wp24, p[f'bp{l}'].reshape(1, -1),
             p[f'Wms{l}'], p[f'Wmd{l}']))
        if use_sc:
            gsp, gdp = _sc_gather_pair(gpt, src, gpt, dst)
            asrc, adst = _sc_gather_pair(ams, src, amd, dst)
        else:
            gsp = jnp.take(gpt, src, axis=0)
            gdp = jnp.take(gpt, dst, axis=0)
            asrc = jnp.take(ams, src, axis=0)
            adst = jnp.take(amd, dst, axis=0)
        wmp16 = jnp.concatenate([p[f'Wmp{l}'], jnp.zeros((8, 128), f32)], axis=0)
        msg0, msg1 = _row_call(
            _message_body, e, te, [32, 32, 128, 128, 128],
            [(32, 16), (128, 128), (16, 128), (1, 128), (128, 256), (1, 256)],
            [128, 128],
            (gsp, gdp, asrc, adst, z, jnp.asarray(_M4), p[f'Wmz{l}'], wmp16,
             p[f'bm{l}'].reshape(1, -1), p[f'Wm2{l}'], p[f'bm2{l}'].reshape(1, -1)))
        if use_sc:
            agg0, agg1 = _sc_segsum_halves(msg0, msg1, dst2d, zrows, n)
        else:
            agg0 = jax.ops.segment_sum(msg0, dst, num_segments=n)
            agg1 = jax.ops.segment_sum(msg1, dst, num_segments=n)
        wn = p[f'Wn{l}']
        s, aes, aed = _row_call(
            _node_post_body, n, tn, [256, 128, 128, 16, 8],
            [(256, 256), (128, 256), (128, 256), (1, 256), (1, 256), (1, 256),
             (256, 128), (256, 128)],
            [256, 128, 128],
            (s, agg0, agg1, deg16, mask8, wn[:256], wn[256:384], wn[384:512],
             p[f'bn{l}'].reshape(1, -1),
             p[f'lns_g{l}'].reshape(1, -1), p[f'lns_b{l}'].reshape(1, -1),
             p[f'Wes{l}'], p[f'Wed{l}']))
        if l < num_layers - 1:
            if use_sc:
                esrc, edst = _sc_gather_pair(aes, src, aed, dst)
            else:
                esrc = jnp.take(aes, src, axis=0)
                edst = jnp.take(aed, dst, axis=0)
            z = _row_call(
                _edge_update_body, e, te, [128, 128, 128],
                [(128, 128), (1, 128), (1, 128), (1, 128)], [128],
                (esrc, edst, z, p[f'Wez{l}'], p[f'be{l}'].reshape(1, -1),
                 p[f'lnz_g{l}'].reshape(1, -1), p[f'lnz_b{l}'].reshape(1, -1)))

    mu, logvar = _row_call(
        _final_body, n, tn, [256],
        [(256, 256), (1, 256), (256, 256), (1, 256)], [256, 256],
        (s, p['Wmu'], p['bmu'].reshape(1, -1), p['Wlv'], p['blv'].reshape(1, -1)))
    return mu, logvar


# speedup vs baseline: 3.4867x; 1.0067x over previous
import functools
import numpy as np
import jax
import jax.numpy as jnp
from jax import lax
from jax.experimental import pallas as pl
from jax.experimental.pallas import tpu as pltpu
from jax.experimental.pallas import tpu_sc as plsc

_NC = 2
_NS = 16
_NW = _NC * _NS
_CH = 200

HI = jax.lax.Precision.HIGHEST

NUM_RBF = 16
N_PT = 8


def _tile(n, prefs=(2000, 1000, 500, 250, 200, 100, 40, 8)):
    for t in prefs:
        if n % t == 0 and t % 8 == 0:
            return t
    return n


def _edge_geom_consts():
    M1 = np.zeros((16, 80), np.float32)
    M2 = np.zeros((16, 80), np.float32)
    M3 = np.zeros((80, 32), np.float32)
    for a in range(5):
        for b in range(5):
            for c in range(3):
                q = (a * 5 + b) * 3 + c
                M1[3 * a + c, q] = 1.0
                M2[3 * b + c, q] = 1.0
                M3[q, a * 5 + b] = 1.0
    S = np.zeros((32, 416), np.float32)
    for p in range(25):
        for k in range(16):
            S[p, 16 * p + k] = 1.0
    mu = np.linspace(2.0, 22.0, NUM_RBF).astype(np.float32)
    MU = np.zeros((416,), np.float32)
    for p in range(25):
        MU[16 * p:16 * p + 16] = mu
    SIG = np.ones((416,), np.float32) * ((22.0 - 2.0) / NUM_RBF)
    PH = np.zeros((16,), np.float32)
    PH[8:] = np.pi / 2
    return M1, M2, M3, S, MU, SIG, PH


_M1, _M2, _M3, _S, _MU, _SIG, _PH = _edge_geom_consts()

_M4 = np.zeros((32, 16), np.float32)
for _p in range(N_PT):
    for _c in range(3):
        _M4[3 * _p + _c, _p] = 1.0

_MP = np.zeros((24, 72), np.float32)
_MR = np.zeros((16, 72), np.float32)
_MS = np.zeros((72, 32), np.float32)
_MT = np.zeros((16, 32), np.float32)
for _p in range(N_PT):
    for _i in range(3):
        for _j in range(3):
            _q = 9 * _p + 3 * _i + _j
            _MP[3 * _p + _j, _q] = 1.0
            _MR[3 * _i + _j, _q] = 1.0
            _MS[_q, 3 * _p + _i] = 1.0
        _MT[9 + _i, 3 * _p + _i] = 1.0


def _ln(x, g, b, eps=1e-5):
    m = jnp.mean(x, axis=-1, keepdims=True)
    xc = x - m
    v = jnp.mean(xc * xc, axis=-1, keepdims=True)
    return xc / jnp.sqrt(v + eps) * g + b


def _edge_embed_body(ebs, ebd, ang, m1, m2, m3, ssel, muv, ph,
                     w0a, w0p, b0, w1, b1, w2, b2, g, beta, z_out):
    t1 = jnp.dot(ebs[...], m1[...], precision=HI)
    t2 = jnp.dot(ebd[...], m2[...], precision=HI)
    diff = t1 - t2 + 1e-8
    d2 = jnp.dot(diff * diff, m3[...], precision=HI)
    d = jnp.sqrt(d2)
    dbig = jnp.dot(d, ssel[...], precision=HI)
    x = jnp.exp(-(((dbig - muv[...]) * 0.8) ** 2))
    pe = jnp.cos(ang[...] - ph[...])
    h = jax.nn.relu(jnp.dot(x, w0a[...]) + jnp.dot(pe, w0p[...]) + b0[...])
    h = jax.nn.relu(jnp.dot(h, w1[...]) + b1[...])
    z_out[...] = _ln(jnp.dot(h, w2[...]) + b2[...], g[...], beta[...])


def _message_body(gsp, gdp, asrc, adst, z, m4, wmz, wmp, bm, wm2, bm2,
                  msg0_out, msg1_out):
    diff = gdp[...] - gsp[...] + 1e-8
    pd2 = jnp.dot(diff * diff, m4[...], precision=HI)
    pd = jnp.sqrt(pd2)
    m = jax.nn.relu(asrc[...] + adst[...] + jnp.dot(z[...], wmz[...])
                    + jnp.dot(pd, wmp[...]) + bm[...])
    msg = jnp.dot(m, wm2[...]) + bm2[...]
    msg0_out[...] = msg[:, 0:128]
    msg1_out[...] = msg[:, 128:256]


def _edge_update_body(esrc, edst, z, wez, be, g, beta, z_out):
    eupd = jax.nn.relu(edst[...] + esrc[...] + jnp.dot(z[...], wez[...]) + be[...])
    z_out[...] = _ln(z[...] + eupd, g[...], beta[...])


def _node_prep_body(ni, bbf, rig, w0, b0, w1, b1, w2, b2, g, beta,
                    s_out, bb5_out, rt_out):
    h = jax.nn.relu(jnp.dot(ni[...], w0[...]) + b0[...])
    h = jax.nn.relu(jnp.dot(h, w1[...]) + b1[...])
    s_out[...] = _ln(jnp.dot(h, w2[...]) + b2[...], g[...], beta[...])
    bf = bbf[...]
    b = bf[:, 3:6] - bf[:, 0:3]
    cv = bf[:, 6:9] - bf[:, 3:6]
    a = jnp.concatenate([
        b[:, 1:2] * cv[:, 2:3] - b[:, 2:3] * cv[:, 1:2],
        b[:, 2:3] * cv[:, 0:1] - b[:, 0:1] * cv[:, 2:3],
        b[:, 0:1] * cv[:, 1:2] - b[:, 1:2] * cv[:, 0:1],
    ], axis=1)
    cb = -0.58273431 * a + 0.56802827 * b - 0.54067466 * cv + bf[:, 3:6]
    zcol = jnp.zeros_like(cb[:, 0:1])
    bb5_out[...] = jnp.concatenate([bf, cb, zcol], axis=1)
    q = rig[:, 0:4]
    qn = q / (jnp.sqrt(jnp.sum(q * q, axis=1, keepdims=True)) + 1e-8)
    w = qn[:, 0:1]; x = qn[:, 1:2]; y = qn[:, 2:3]; zc = qn[:, 3:4]
    rt_out[...] = jnp.concatenate([
        1 - 2 * (y * y + zc * zc), 2 * (x * y - zc * w), 2 * (x * zc + y * w),
        2 * (x * y + zc * w), 1 - 2 * (x * x + zc * zc), 2 * (y * zc - x * w),
        2 * (x * zc - y * w), 2 * (y * zc + x * w), 1 - 2 * (x * x + y * y),
        rig[:, 4:7], zcol, zcol, zcol, zcol,
    ], axis=1)


def _node_pre_body(s, rt, mp, mr, ms, mt, wp, bp, wms, wmd,
                   gpt_out, ams_out, amd_out):
    pts = jnp.dot(s[...], wp[...]) + bp[...]
    pe = jnp.dot(pts, mp[...], precision=HI)
    re = jnp.dot(rt[...], mr[...], precision=HI)
    gpf = jnp.dot(pe * re, ms[...], precision=HI) + jnp.dot(rt[...], mt[...], precision=HI)
    gpt_out[...] = gpf
    ams_out[...] = jnp.dot(s[...], wms[...])
    amd_out[...] = jnp.dot(s[...], wmd[...])


def _node_post_body(s, agg0, agg1, deg16, mask8, wns, wna0, wna1, bn, g, beta,
                    wes, wed, s_out, aes_out, aed_out):
    deg = jnp.clip(deg16[:, 0:1], 1.0, None)
    upd = jax.nn.relu(jnp.dot(s[...], wns[...]) + jnp.dot(agg0[...] / deg, wna0[...])
                      + jnp.dot(agg1[...] / deg, wna1[...]) + bn[...])
    s2 = _ln(s[...] + upd, g[...], beta[...]) * mask8[:, 0:1]
    s_out[...] = s2
    aes_out[...] = jnp.dot(s2, wes[...])
    aed_out[...] = jnp.dot(s2, wed[...])


def _final_body(s, wmu, bmu, wlv, blv, mu_out, lv_out):
    mu_out[...] = jnp.dot(s[...], wmu[...]) + bmu[...]
    lv_out[...] = jnp.dot(s[...], wlv[...]) + blv[...]


def _sc_mesh():
    return plsc.VectorSubcoreMesh(core_axis_name="c", subcore_axis_name="s")


_SC_CP = pltpu.CompilerParams(use_tc_tiling_on_sc=False)


def _sc_gather_pair(table_a, idx_a, table_b, idx_b):
    e = idx_a.shape[0]
    d = table_a.shape[1]
    per_w = e // _NW
    nch = per_w // _CH

    @functools.partial(
        pl.kernel, mesh=_sc_mesh(), compiler_params=_SC_CP,
        out_type=[jax.ShapeDtypeStruct((e, d), jnp.float32),
                  jax.ShapeDtypeStruct((e, d), jnp.float32)],
        scratch_types=[pltpu.VMEM((2, _CH), jnp.int32),
                       pltpu.VMEM((2, _CH, d), jnp.float32),
                       pltpu.VMEM((2, _CH), jnp.int32),
                       pltpu.VMEM((2, _CH, d), jnp.float32),
                       pltpu.SemaphoreType.DMA,
                       pltpu.SemaphoreType.DMA,
                       pltpu.SemaphoreType.DMA,
                       pltpu.SemaphoreType.DMA],
    )
    def k(ta, ia, tb, ib, oa, ob, iva, rva, ivb, rvb, sa0, sa1, sb0, sb1):
        wid = lax.axis_index("s") * _NC + lax.axis_index("c")
        base = wid * per_w
        sems = ((sa0, sb0), (sa1, sb1))

        def start(i, sl):
            b = base + i * _CH
            sema, semb = sems[sl]
            pltpu.sync_copy(ia.at[pl.ds(b, _CH)], iva.at[sl])
            pltpu.async_copy(ta.at[iva.at[sl]], rva.at[sl], sema)
            pltpu.sync_copy(ib.at[pl.ds(b, _CH)], ivb.at[sl])
            pltpu.async_copy(tb.at[ivb.at[sl]], rvb.at[sl], semb)

        def drain(i, sl):
            b = base + i * _CH
            sema, semb = sems[sl]
            pltpu.make_async_copy(ta.at[iva.at[sl]], rva.at[sl], sema).wait()
            pltpu.sync_copy(rva.at[sl], oa.at[pl.ds(b, _CH)])
            pltpu.make_async_copy(tb.at[ivb.at[sl]], rvb.at[sl], semb).wait()
            pltpu.sync_copy(rvb.at[sl], ob.at[pl.ds(b, _CH)])

        start(0, 0)
        for i in range(nch - 1):
            start(i + 1, (i + 1) % 2)
            drain(i, i % 2)
        drain(nch - 1, (nch - 1) % 2)

    return k(table_a, idx_a, table_b, idx_b)


def _sc_segsum_halves(msg0, msg1, dst2d, zrows, n):
    e = msg0.shape[0]
    nch_tot = e // _CH
    nzch = n // _CH

    @functools.partial(
        pl.kernel, mesh=_sc_mesh(), compiler_params=_SC_CP,
        out_type=[jax.ShapeDtypeStruct((n, 128), jnp.float32),
                  jax.ShapeDtypeStruct((n, 128), jnp.float32)],
        scratch_types=[pltpu.VMEM((1, _CH), jnp.int32),
                       pltpu.VMEM((_CH, 128), jnp.float32),
                       pltpu.VMEM_SHARED((n, 128), jnp.float32),
                       pltpu.SemaphoreType.DMA],
    )
    def k(m0, m1, dd, zr, o0, o1, idxv, rows, acc, sem):
        cid = lax.axis_index("c")
        sid = lax.axis_index("s")

        @pl.loop(0, nzch)
        def _(i):
            @pl.when((i % _NS) == sid)
            def _():
                pltpu.sync_copy(zr, acc.at[pl.ds(i * _CH, _CH)])

        plsc.subcore_barrier()

        def scan(mref):
            @pl.loop(0, nch_tot)
            def _(i):
                @pl.when((i % _NS) == sid)
                def _():
                    pltpu.sync_copy(dd.at[pl.ds(i, 1)], idxv)
                    pltpu.sync_copy(mref.at[pl.ds(i * _CH, _CH)], rows)
                    pltpu.sync_copy(rows, acc.at[idxv.at[0]], add=True)

        @pl.when(cid == 0)
        def _():
            scan(m0)

        @pl.when(cid == 1)
        def _():
            scan(m1)

        plsc.subcore_barrier()

        def dump(oref):
            @pl.loop(0, nzch)
            def _(i):
                @pl.when((i % _NS) == sid)
                def _():
                    pltpu.sync_copy(acc.at[pl.ds(i * _CH, _CH)],
                                    oref.at[pl.ds(i * _CH, _CH)])

        @pl.when(cid == 0)
        def _():
            dump(o0)

        @pl.when(cid == 1)
        def _():
            dump(o1)

    return k(msg0, msg1, dst2d, zrows)


def _sc_deg(dst2d, ones16, z16, n):
    nch_tot = (dst2d.shape[0] * dst2d.shape[1]) // _CH
    nzch = n // _CH

    @functools.partial(
        pl.kernel, mesh=_sc_mesh(), compiler_params=_SC_CP,
        out_type=jax.ShapeDtypeStruct((n, 16), jnp.float32),
        scratch_types=[pltpu.VMEM((1, _CH), jnp.int32),
                       pltpu.VMEM((_CH, 16), jnp.float32),
                       pltpu.VMEM_SHARED((n, 16), jnp.float32),
                       pltpu.SemaphoreType.DMA],
    )
    def k(dd, ones_h, zr, out, idxv, onev, acc, sem):
        cid = lax.axis_index("c")
        sid = lax.axis_index("s")

        @pl.when(cid == 0)
        def _():
            pltpu.sync_copy(ones_h, onev)

            @pl.loop(0, nzch)
            def _(i):
                @pl.when((i % _NS) == sid)
                def _():
                    pltpu.sync_copy(zr, acc.at[pl.ds(i * _CH, _CH)])

            plsc.subcore_barrier()

            @pl.loop(0, nch_tot)
            def _(i):
                @pl.when((i % _NS) == sid)
                def _():
                    pltpu.sync_copy(dd.at[pl.ds(i, 1)], idxv)
                    pltpu.sync_copy(onev, acc.at[idxv.at[0]], add=True)

            plsc.subcore_barrier()

            @pl.loop(0, nzch)
            def _(i):
                @pl.when((i % _NS) == sid)
                def _():
                    pltpu.sync_copy(acc.at[pl.ds(i * _CH, _CH)],
                                    out.at[pl.ds(i * _CH, _CH)])

    return k(dst2d, ones16, z16)


def _row_call(body, n_rows, tile, row_widths, const_shapes, out_widths, args):
    grid = (n_rows // tile,)
    in_specs = [pl.BlockSpec((tile, w), lambda i: (i, 0)) for w in row_widths]
    in_specs += [pl.BlockSpec(s, lambda i, _r=len(s): (0,) * _r) for s in const_shapes]
    out_specs = [pl.BlockSpec((tile, w), lambda i: (i, 0)) for w in out_widths]
    out_shape = [jax.ShapeDtypeStruct((n_rows, w), jnp.float32) for w in out_widths]
    if len(out_widths) == 1:
        out_specs = out_specs[0]
        out_shape = out_shape[0]
    return pl.pallas_call(
        body, grid=grid, in_specs=in_specs, out_specs=out_specs,
        out_shape=out_shape)(*args)


def kernel(node_in, bb, rigids_7, res_mask, edge_index, params):
    p = params
    src = edge_index[1]
    dst = edge_index[0]
    n = node_in.shape[0]
    e = src.shape[0]
    tn = _tile(n)
    te = _tile(e)
    f32 = jnp.float32

    bbf = bb.reshape(n, 12)
    rig = rigids_7
    w0p_shapes = [(node_in.shape[1], 512), (1, 512), (512, 512), (1, 512),
                  (512, 256), (1, 256), (1, 256), (1, 256)]
    s0, bb5f, rt = _row_call(
        _node_prep_body, n, tn, [node_in.shape[1], 12, 7], w0p_shapes,
        [256, 16, 16],
        (node_in, bbf, rig,
         p['en_W0'], p['en_b0'].reshape(1, -1), p['en_W1'], p['en_b1'].reshape(1, -1),
         p['en_W2'], p['en_b2'].reshape(1, -1), p['en_g'].reshape(1, -1),
         p['en_beta'].reshape(1, -1)))

    use_sc = (e % (_NW * _CH) == 0) and (n % _CH == 0) and (n * 128 * 4 <= 6 * 2 ** 20)
    if use_sc:
        dst2d = dst.reshape(e // _CH, _CH)
        zrows = jnp.zeros((_CH, 128), f32)
        zrows16 = jnp.zeros((_CH, 16), f32)
        ones16 = jnp.concatenate([jnp.ones((_CH, 1), f32),
                                  jnp.zeros((_CH, 15), f32)], axis=1)
        deg16 = _sc_deg(dst2d, ones16, zrows16, n)
        ebs, ebd = _sc_gather_pair(bb5f, src, bb5f, dst)
    else:
        deg16 = jnp.broadcast_to(
            jax.ops.segment_sum(jnp.ones((e,), f32), dst, num_segments=n)[:, None],
            (n, 16))
        ebs = jnp.take(bb5f, src, axis=0)
        ebd = jnp.take(bb5f, dst, axis=0)
    dif = (dst - src).astype(f32)
    freq = jnp.exp(jnp.arange(0, 16, 2, dtype=f32) * (-np.log(10000.0) / 16))
    freq2 = jnp.concatenate([freq, freq])
    ang = dif[:, None] * freq2[None, :]
    w0a = jnp.concatenate([p['ee_W0'][:400], jnp.zeros((16, 256), f32)], axis=0)
    w0pe = p['ee_W0'][400:416]
    z = _row_call(
        _edge_embed_body, e, te, [16, 16, 16],
        [(16, 80), (16, 80), (80, 32), (32, 416), (1, 416), (1, 16),
         (416, 256), (16, 256), (1, 256), (256, 256), (1, 256), (256, 128),
         (1, 128), (1, 128), (1, 128)],
        [128],
        (ebs, ebd, ang, jnp.asarray(_M1), jnp.asarray(_M2), jnp.asarray(_M3),
         jnp.asarray(_S), jnp.asarray(_MU).reshape(1, -1),
         jnp.asarray(_PH).reshape(1, -1),
         w0a, w0pe, p['ee_b0'].reshape(1, -1), p['ee_W1'],
         p['ee_b1'].reshape(1, -1), p['ee_W2'], p['ee_b2'].reshape(1, -1),
         p['ee_g'].reshape(1, -1), p['ee_beta'].reshape(1, -1)))

    mask8 = jnp.broadcast_to(res_mask[:, None], (n, 8))

    s = s0
    num_layers = 2
    for l in range(num_layers):
        wp24 = p[f'Wp{l}']
        gpt, ams, amd = _row_call(
            _node_pre_body, n, tn, [256, 16],
            [(24, 72), (16, 72), (72, 32), (16, 32),
             (256, 24), (1, 24), (256, 128), (256, 128)], [32, 128, 128],
            (s, rt, jnp.asarray(_MP), jnp.asarray(_MR), jnp.asarray(_MS),
             jnp.asarray(_MT), wp24, p[f'bp{l}'].reshape(1, -1),
             p[f'Wms{l}'], p[f'Wmd{l}']))
        if use_sc:
            gsp, gdp = _sc_gather_pair(gpt, src, gpt, dst)
            asrc, adst = _sc_gather_pair(ams, src, amd, dst)
        else:
            gsp = jnp.take(gpt, src, axis=0)
            gdp = jnp.take(gpt, dst, axis=0)
            asrc = jnp.take(ams, src, axis=0)
            adst = jnp.take(amd, dst, axis=0)
        wmp16 = jnp.concatenate([p[f'Wmp{l}'], jnp.zeros((8, 128), f32)], axis=0)
        msg0, msg1 = _row_call(
            _message_body, e, te, [32, 32, 128, 128, 128],
            [(32, 16), (128, 128), (16, 128), (1, 128), (128, 256), (1, 256)],
            [128, 128],
            (gsp, gdp, asrc, adst, z, jnp.asarray(_M4), p[f'Wmz{l}'], wmp16,
             p[f'bm{l}'].reshape(1, -1), p[f'Wm2{l}'], p[f'bm2{l}'].reshape(1, -1)))
        if use_sc:
            agg0, agg1 = _sc_segsum_halves(msg0, msg1, dst2d, zrows, n)
        else:
            agg0 = jax.ops.segment_sum(msg0, dst, num_segments=n)
            agg1 = jax.ops.segment_sum(msg1, dst, num_segments=n)
        wn = p[f'Wn{l}']
        s, aes, aed = _row_call(
            _node_post_body, n, tn, [256, 128, 128, 16, 8],
            [(256, 256), (128, 256), (128, 256), (1, 256), (1, 256), (1, 256),
             (256, 128), (256, 128)],
            [256, 128, 128],
            (s, agg0, agg1, deg16, mask8, wn[:256], wn[256:384], wn[384:512],
             p[f'bn{l}'].reshape(1, -1),
             p[f'lns_g{l}'].reshape(1, -1), p[f'lns_b{l}'].reshape(1, -1),
             p[f'Wes{l}'], p[f'Wed{l}']))
        if l < num_layers - 1:
            if use_sc:
                esrc, edst = _sc_gather_pair(aes, src, aed, dst)
            else:
                esrc = jnp.take(aes, src, axis=0)
                edst = jnp.take(aed, dst, axis=0)
            z = _row_call(
                _edge_update_body, e, te, [128, 128, 128],
                [(128, 128), (1, 128), (1, 128), (1, 128)], [128],
                (esrc, edst, z, p[f'Wez{l}'], p[f'be{l}'].reshape(1, -1),
                 p[f'lnz_g{l}'].reshape(1, -1), p[f'lnz_b{l}'].reshape(1, -1)))

    mu, logvar = _row_call(
        _final_body, n, tn, [256],
        [(256, 256), (1, 256), (256, 256), (1, 256)], [256, 256],
        (s, p['Wmu'], p['bmu'].reshape(1, -1), p['Wlv'], p['blv'].reshape(1, -1)))
    return mu, logvar
